# Initial kernel scaffold; baseline (speedup 1.0000x reference)
import jax
import jax.numpy as jnp
from jax.experimental import pallas as pl


def kernel(x, edge_index, W, b):
    raise NotImplementedError("write your pallas kernel here")


# speedup vs baseline: 9.0953x; 9.0953x over previous
import jax
import jax.numpy as jnp
from jax import lax
from jax.experimental import pallas as pl
from jax.experimental.pallas import tpu as pltpu
from jax.experimental.pallas import tpu_sc as plsc

N = 10000
D = 128
NC = 2
NS = 16
NW = NC * NS
L = 16
NP = 10240
KB = 128
EPW = 10240
NB = EPW // KB
EP = NW * EPW
NBUF = 2
GA = 1
RPT = NP // NS
RB = 2000

HALF = NP // NC
EPT = EP // NS
NB2 = EPT // KB
DUMMY_DST = 1 << 20
ACC_ROWS = HALF + KB
ZPT = ACC_ROWS // NS
WPT = HALF // NS

_mesh = plsc.VectorSubcoreMesh(core_axis_name="c", subcore_axis_name="s")


def _deg_body(dst_hbm, deg_out, didx, ones_v, zbuf, deg_sh):
    c = lax.axis_index("c")
    s = lax.axis_index("s")
    wid = s * NC + c
    pltpu.sync_copy(dst_hbm.at[wid], didx)
    for k in range(KB // L):
        ones_v[pl.ds(k * L, L)] = jnp.ones((L,), jnp.float32)
    for k in range(RPT // L):
        zbuf[pl.ds(k * L, L)] = jnp.zeros((L,), jnp.float32)
    pltpu.sync_copy(zbuf, deg_sh.at[pl.ds(s * RPT, RPT)])
    plsc.subcore_barrier()

    def body(j, carry):
        pltpu.sync_copy(ones_v, deg_sh.at[didx.at[j]], add=True)
        return carry

    lax.fori_loop(0, NB, body, 0)
    plsc.subcore_barrier()
    pltpu.sync_copy(deg_sh.at[pl.ds(s * RPT, RPT)],
                    deg_out.at[c, pl.ds(s * RPT, RPT)])


def _msg_body(src_hbm, dst_hbm, y_hbm, acc_out,
              sidx, didx, gbuf, acc_sh, gsem, ssem):
    c = lax.axis_index("c")
    s = lax.axis_index("s")
    pltpu.sync_copy(src_hbm.at[s], sidx)
    pltpu.sync_copy(dst_hbm.at[s], didx)

    lo = c * HALF
    dummy = HALF + lax.iota(jnp.int32, L) * 8

    def remap(jb, carry):
        for k in range(KB // L):
            dv = didx[jb, pl.ds(k * L, L)]
            lv = dv - lo
            m = (lv >= 0) & (lv < HALF)
            didx[jb, pl.ds(k * L, L)] = jnp.where(m, lv, dummy)
        return carry

    lax.fori_loop(0, NB2, remap, 0)

    def zrow(r, carry):
        for k in range(D // L):
            gbuf[0, r, pl.ds(k * L, L)] = jnp.zeros((L,), jnp.float32)
        return carry

    lax.fori_loop(0, KB, zrow, 0)
    zbase = s * ZPT
    pltpu.sync_copy(gbuf.at[0], acc_sh.at[pl.ds(zbase, KB)])
    pltpu.sync_copy(gbuf.at[0], acc_sh.at[pl.ds(zbase + KB, KB)])
    pltpu.sync_copy(gbuf.at[0, pl.ds(0, ZPT - 2 * KB)],
                    acc_sh.at[pl.ds(zbase + 2 * KB, ZPT - 2 * KB)])
    plsc.subcore_barrier()

    def gather_start(j, bb):
        pltpu.async_copy(y_hbm.at[sidx.at[j]], gbuf.at[bb], gsem.at[bb])

    def gather_wait(bb):
        pltpu.make_async_copy(y_hbm.at[sidx.at[0]], gbuf.at[bb],
                              gsem.at[bb]).wait()

    def scat_start(j, bb):
        pltpu.async_copy(gbuf.at[bb], acc_sh.at[didx.at[j]], ssem.at[bb],
                         add=True)

    def scat_wait(bb):
        pltpu.make_async_copy(gbuf.at[bb], acc_sh.at[didx.at[0]],
                              ssem.at[bb]).wait()

    for j in range(GA):
        gather_start(j, j % NBUF)

    def group(g, carry):
        for b in range(NBUF):
            j = g * NBUF + b
            nj = j + GA
            nb = (b + GA) % NBUF

            @pl.when(nj < NB2)
            def _():
                @pl.when(nj >= NBUF)
                def _():
                    scat_wait(nb)

                gather_start(nj, nb)

            gather_wait(b)
            scat_start(j, b)
        return carry

    lax.fori_loop(0, NB2 // NBUF, group, 0)
    for b in range(NBUF):
        scat_wait(b)
    plsc.subcore_barrier()
    pltpu.sync_copy(acc_sh.at[pl.ds(s * WPT, WPT)],
                    acc_out.at[c, pl.ds(s * WPT, WPT)])


def _mm_body(x_ref, w_ref, dg_ref, y_ref):
    dg = dg_ref[...]
    dinv = lax.rsqrt(dg[:, 0:1] + dg[:, 1:2] + 1.0)
    y_ref[...] = jnp.dot(x_ref[...], w_ref[...],
                         preferred_element_type=jnp.float32) * dinv


def _fin_body(acc_ref, y_ref, dg_ref, b_ref, o_ref):
    dg = dg_ref[...]
    dinv = lax.rsqrt(dg[:, 0:1] + dg[:, 1:2] + 1.0)
    o_ref[...] = (acc_ref[...] + y_ref[...]) * dinv + b_ref[...]


def kernel(x, edge_index, W, b):
    pad = EP - edge_index.shape[1]
    dst_a = jnp.concatenate(
        [edge_index[1], jnp.full((pad,), N, jnp.int32)]).reshape(NW, NB, KB)
    src_c = jnp.concatenate(
        [edge_index[0], jnp.zeros((pad,), jnp.int32)]).reshape(NS, NB2, KB)
    dst_c = jnp.concatenate(
        [edge_index[1],
         jnp.full((pad,), DUMMY_DST, jnp.int32)]).reshape(NS, NB2, KB)

    deg_fn = pl.kernel(
        _deg_body,
        out_type=jax.ShapeDtypeStruct((NC, NP), jnp.float32),
        mesh=_mesh,
        scratch_types=[
            pltpu.VMEM((NB, KB), jnp.int32),
            pltpu.VMEM((KB,), jnp.float32),
            pltpu.VMEM((RPT,), jnp.float32),
            pltpu.VMEM_SHARED((NP,), jnp.float32),
        ],
    )
    deg = deg_fn(dst_a)
    deg_t = deg.T

    y = pl.pallas_call(
        _mm_body,
        grid=(N // RB,),
        in_specs=[
            pl.BlockSpec((RB, D), lambda i: (i, 0)),
            pl.BlockSpec((D, D), lambda i: (0, 0)),
            pl.BlockSpec((RB, NC), lambda i: (i, 0)),
        ],
        out_specs=pl.BlockSpec((RB, D), lambda i: (i, 0)),
        out_shape=jax.ShapeDtypeStruct((N, D), jnp.float32),
    )(x, W, deg_t)

    msg_fn = pl.kernel(
        _msg_body,
        out_type=jax.ShapeDtypeStruct((NC, HALF, D), jnp.float32),
        mesh=_mesh,
        scratch_types=[
            pltpu.VMEM((NB2, KB), jnp.int32),
            pltpu.VMEM((NB2, KB), jnp.int32),
            pltpu.VMEM((NBUF, KB, D), jnp.float32),
            pltpu.VMEM_SHARED((ACC_ROWS, D), jnp.float32),
            pltpu.SemaphoreType.DMA((NBUF,)),
            pltpu.SemaphoreType.DMA((NBUF,)),
        ],
    )
    acc = msg_fn(src_c, dst_c, y).reshape(NP, D)

    out = pl.pallas_call(
        _fin_body,
        grid=(N // RB,),
        in_specs=[
            pl.BlockSpec((RB, D), lambda i: (i, 0)),
            pl.BlockSpec((RB, D), lambda i: (i, 0)),
            pl.BlockSpec((RB, NC), lambda i: (i, 0)),
            pl.BlockSpec((1, D), lambda i: (0, 0)),
        ],
        out_specs=pl.BlockSpec((RB, D), lambda i: (i, 0)),
        out_shape=jax.ShapeDtypeStruct((N, D), jnp.float32),
    )(acc, y, deg_t, b.reshape(1, D))
    return out


# speedup vs baseline: 9.5193x; 1.0466x over previous
import jax
import jax.numpy as jnp
from jax import lax
from jax.experimental import pallas as pl
from jax.experimental.pallas import tpu as pltpu
from jax.experimental.pallas import tpu_sc as plsc

N = 10000
D = 128
NC = 2
NS = 16
NW = NC * NS
L = 16
NP = 10240
KB = 128
EPW = 10240
NB = EPW // KB
EP = NW * EPW
NBUF = 2
GA = 1
RPT = NP // NS
RB = 2000

HALF = NP // NC
EPT = EP // NS
NB2 = EPT // KB
DUMMY_DST = 1 << 20
ACC_ROWS = HALF + KB
ZPT = ACC_ROWS // NS
WPT = HALF // NS

_mesh = plsc.VectorSubcoreMesh(core_axis_name="c", subcore_axis_name="s")


def _deg_body(dst_hbm, deg_out, didx, ones_v, zbuf, deg_sh):
    c = lax.axis_index("c")
    s = lax.axis_index("s")
    wid = s * NC + c
    pltpu.sync_copy(dst_hbm.at[wid], didx)
    for k in range(KB // L):
        ones_v[pl.ds(k * L, L)] = jnp.ones((L,), jnp.float32)
    for k in range(RPT // L):
        zbuf[pl.ds(k * L, L)] = jnp.zeros((L,), jnp.float32)
    pltpu.sync_copy(zbuf, deg_sh.at[pl.ds(s * RPT, RPT)])
    plsc.subcore_barrier()

    def body(j, carry):
        pltpu.sync_copy(ones_v, deg_sh.at[didx.at[j]], add=True)
        return carry

    lax.fori_loop(0, NB, body, 0)
    plsc.subcore_barrier()
    pltpu.sync_copy(deg_sh.at[pl.ds(s * RPT, RPT)],
                    deg_out.at[c, pl.ds(s * RPT, RPT)])


def _msg_body(src_hbm, dst_hbm, y_hbm, acc_out,
              sidx, didx, gbuf, acc_sh, gsem, ssem):
    c = lax.axis_index("c")
    s = lax.axis_index("s")
    pltpu.sync_copy(src_hbm.at[s], sidx)
    pltpu.sync_copy(dst_hbm.at[s], didx)

    lo = c * HALF
    iota = lax.iota(jnp.int32, L)

    def remap(jb, carry):
        for k in range(KB // L):
            dv = didx[jb, pl.ds(k * L, L)]
            lv = dv - lo
            m = (lv >= 0) & (lv < HALF)
            dummy = HALF + ((iota * 8 + jb + k) & (KB - 1))
            didx[jb, pl.ds(k * L, L)] = jnp.where(m, lv, dummy)
        return carry

    lax.fori_loop(0, NB2, remap, 0)

    def zrow(r, carry):
        for k in range(D // L):
            gbuf[0, r, pl.ds(k * L, L)] = jnp.zeros((L,), jnp.float32)
        return carry

    lax.fori_loop(0, KB, zrow, 0)
    zbase = s * ZPT
    pltpu.sync_copy(gbuf.at[0], acc_sh.at[pl.ds(zbase, KB)])
    pltpu.sync_copy(gbuf.at[0], acc_sh.at[pl.ds(zbase + KB, KB)])
    pltpu.sync_copy(gbuf.at[0, pl.ds(0, ZPT - 2 * KB)],
                    acc_sh.at[pl.ds(zbase + 2 * KB, ZPT - 2 * KB)])
    plsc.subcore_barrier()

    def gather_start(j, bb):
        pltpu.async_copy(y_hbm.at[sidx.at[j]], gbuf.at[bb], gsem.at[bb])

    def gather_wait(bb):
        pltpu.make_async_copy(y_hbm.at[sidx.at[0]], gbuf.at[bb],
                              gsem.at[bb]).wait()

    def scat_start(j, bb):
        pltpu.async_copy(gbuf.at[bb], acc_sh.at[didx.at[j]], ssem.at[bb],
                         add=True)

    def scat_wait(bb):
        pltpu.make_async_copy(gbuf.at[bb], acc_sh.at[didx.at[0]],
                              ssem.at[bb]).wait()

    for j in range(GA):
        gather_start(j, j % NBUF)

    def group(g, carry):
        for b in range(NBUF):
            j = g * NBUF + b
            nj = j + GA
            nb = (b + GA) % NBUF

            @pl.when(nj < NB2)
            def _():
                @pl.when(nj >= NBUF)
                def _():
                    scat_wait(nb)

                gather_start(nj, nb)

            gather_wait(b)
            scat_start(j, b)
        return carry

    lax.fori_loop(0, NB2 // NBUF, group, 0)
    for b in range(NBUF):
        scat_wait(b)
    plsc.subcore_barrier()
    pltpu.sync_copy(acc_sh.at[pl.ds(s * WPT, WPT)],
                    acc_out.at[c, pl.ds(s * WPT, WPT)])


def _mm_body(x_ref, w_ref, dg_ref, y_ref):
    dg = dg_ref[...]
    dinv = lax.rsqrt(dg[:, 0:1] + dg[:, 1:2] + 1.0)
    y_ref[...] = jnp.dot(x_ref[...], w_ref[...],
                         preferred_element_type=jnp.float32) * dinv


def _fin_body(acc_ref, y_ref, dg_ref, b_ref, o_ref):
    dg = dg_ref[...]
    dinv = lax.rsqrt(dg[:, 0:1] + dg[:, 1:2] + 1.0)
    o_ref[...] = (acc_ref[...] + y_ref[...]) * dinv + b_ref[...]


def kernel(x, edge_index, W, b):
    pad = EP - edge_index.shape[1]
    dst_a = jnp.concatenate(
        [edge_index[1], jnp.full((pad,), N, jnp.int32)]).reshape(NW, NB, KB)
    src_c = jnp.concatenate(
        [edge_index[0], jnp.zeros((pad,), jnp.int32)]).reshape(NS, NB2, KB)
    dst_c = jnp.concatenate(
        [edge_index[1],
         jnp.full((pad,), DUMMY_DST, jnp.int32)]).reshape(NS, NB2, KB)

    deg_fn = pl.kernel(
        _deg_body,
        out_type=jax.ShapeDtypeStruct((NC, NP), jnp.float32),
        mesh=_mesh,
        scratch_types=[
            pltpu.VMEM((NB, KB), jnp.int32),
            pltpu.VMEM((KB,), jnp.float32),
            pltpu.VMEM((RPT,), jnp.float32),
            pltpu.VMEM_SHARED((NP,), jnp.float32),
        ],
    )
    deg = deg_fn(dst_a)
    deg_t = deg.T

    y = pl.pallas_call(
        _mm_body,
        grid=(N // RB,),
        in_specs=[
            pl.BlockSpec((RB, D), lambda i: (i, 0)),
            pl.BlockSpec((D, D), lambda i: (0, 0)),
            pl.BlockSpec((RB, NC), lambda i: (i, 0)),
        ],
        out_specs=pl.BlockSpec((RB, D), lambda i: (i, 0)),
        out_shape=jax.ShapeDtypeStruct((N, D), jnp.float32),
    )(x, W, deg_t)

    msg_fn = pl.kernel(
        _msg_body,
        out_type=jax.ShapeDtypeStruct((NC, HALF, D), jnp.float32),
        mesh=_mesh,
        scratch_types=[
            pltpu.VMEM((NB2, KB), jnp.int32),
            pltpu.VMEM((NB2, KB), jnp.int32),
            pltpu.VMEM((NBUF, KB, D), jnp.float32),
            pltpu.VMEM_SHARED((ACC_ROWS, D), jnp.float32),
            pltpu.SemaphoreType.DMA((NBUF,)),
            pltpu.SemaphoreType.DMA((NBUF,)),
        ],
    )
    acc = msg_fn(src_c, dst_c, y).reshape(NP, D)

    out = pl.pallas_call(
        _fin_body,
        grid=(N // RB,),
        in_specs=[
            pl.BlockSpec((RB, D), lambda i: (i, 0)),
            pl.BlockSpec((RB, D), lambda i: (i, 0)),
            pl.BlockSpec((RB, NC), lambda i: (i, 0)),
            pl.BlockSpec((1, D), lambda i: (0, 0)),
        ],
        out_specs=pl.BlockSpec((RB, D), lambda i: (i, 0)),
        out_shape=jax.ShapeDtypeStruct((N, D), jnp.float32),
    )(acc, y, deg_t, b.reshape(1, D))
    return out


# speedup vs baseline: 14.7382x; 1.5482x over previous
import jax
import jax.numpy as jnp
from jax import lax
from jax.experimental import pallas as pl
from jax.experimental.pallas import tpu as pltpu
from jax.experimental.pallas import tpu_sc as plsc

N = 10000
D = 128
NC = 2
NS = 16
NW = NC * NS
L = 16
NP = 10240
KB = 128
EPW = 10240
NB = EPW // KB
EP = NW * EPW
NBUF = 2
GA = 1
RPT = NP // NS
RB = 2000

IR = 4

_mesh = plsc.VectorSubcoreMesh(core_axis_name="c", subcore_axis_name="s")


def _deg_body(dst_hbm, deg_out, didx, ones_v, zbuf, deg_sh):
    c = lax.axis_index("c")
    s = lax.axis_index("s")
    wid = s * NC + c
    pltpu.sync_copy(dst_hbm.at[wid], didx)
    for k in range(KB // L):
        ones_v[pl.ds(k * L, L)] = jnp.ones((L,), jnp.float32)
    for k in range(RPT // L):
        zbuf[pl.ds(k * L, L)] = jnp.zeros((L,), jnp.float32)
    pltpu.sync_copy(zbuf, deg_sh.at[pl.ds(s * RPT, RPT)])
    plsc.subcore_barrier()

    def body(j, carry):
        pltpu.sync_copy(ones_v, deg_sh.at[didx.at[j]], add=True)
        return carry

    lax.fori_loop(0, NB, body, 0)
    plsc.subcore_barrier()
    pltpu.sync_copy(deg_sh.at[pl.ds(s * RPT, RPT)],
                    deg_out.at[c, pl.ds(s * RPT, RPT)])


def _msg_body(src_hbm, dst_hbm, y_hbm, acc_out,
              sring, dring, gbuf, acc_sh, isem, gsem, ssem):
    c = lax.axis_index("c")
    s = lax.axis_index("s")
    wid = s * NC + c

    def zrow(r, carry):
        for k in range(D // L):
            gbuf[0, r, pl.ds(k * L, L)] = jnp.zeros((L,), jnp.float32)
        return carry

    lax.fori_loop(0, KB, zrow, 0)
    for i in range(RPT // KB):
        pltpu.sync_copy(gbuf.at[0], acc_sh.at[pl.ds(s * RPT + i * KB, KB)])
    plsc.subcore_barrier()

    def iload_start(j, r):
        pltpu.async_copy(src_hbm.at[wid, j], sring.at[r], isem.at[r])
        pltpu.async_copy(dst_hbm.at[wid, j], dring.at[r], isem.at[r])

    def iload_wait(r):
        pltpu.make_async_copy(src_hbm.at[wid, 0], sring.at[r],
                              isem.at[r]).wait()
        pltpu.make_async_copy(dst_hbm.at[wid, 0], dring.at[r],
                              isem.at[r]).wait()

    def gather_start(r, bb):
        pltpu.async_copy(y_hbm.at[sring.at[r]], gbuf.at[bb], gsem.at[bb])

    def gather_wait(bb):
        pltpu.make_async_copy(y_hbm.at[sring.at[0]], gbuf.at[bb],
                              gsem.at[bb]).wait()

    def scat_start(r, bb):
        pltpu.async_copy(gbuf.at[bb], acc_sh.at[dring.at[r]], ssem.at[bb],
                         add=True)

    def scat_wait(bb):
        pltpu.make_async_copy(gbuf.at[bb], acc_sh.at[dring.at[0]],
                              ssem.at[bb]).wait()

    iload_start(0, 0)
    iload_start(1, 1)
    iload_wait(0)
    gather_start(0, 0)

    def group(g, carry):
        for u in range(IR):
            j = g * IR + u
            b = u % NBUF

            @pl.when(j >= 1)
            def _():
                scat_wait((u + 1) % NBUF)

            @pl.when(j + 2 < NB)
            def _():
                iload_start(j + 2, (u + 2) % IR)

            @pl.when(j + 1 < NB)
            def _():
                iload_wait((u + 1) % IR)
                gather_start((u + 1) % IR, (u + 1) % NBUF)

            gather_wait(b)
            scat_start(u, b)
        return carry

    lax.fori_loop(0, NB // IR, group, 0)
    scat_wait((NB - 1) % NBUF)
    plsc.subcore_barrier()
    pltpu.sync_copy(acc_sh.at[pl.ds(s * RPT, RPT)],
                    acc_out.at[c, pl.ds(s * RPT, RPT)])


def _mm_body(x_ref, w_ref, dg_ref, y_ref):
    dg = dg_ref[...]
    dinv = lax.rsqrt(dg[:, 0:1] + dg[:, 1:2] + 1.0)
    y_ref[...] = jnp.dot(x_ref[...], w_ref[...],
                         preferred_element_type=jnp.float32) * dinv


def _fin_body(acc_ref, y_ref, dg_ref, b_ref, o_ref):
    dg = dg_ref[...]
    dinv = lax.rsqrt(dg[:, 0:1] + dg[:, 1:2] + 1.0)
    tot = acc_ref[0] + acc_ref[1] + y_ref[...]
    o_ref[...] = tot * dinv + b_ref[...]


def kernel(x, edge_index, W, b):
    pad = EP - edge_index.shape[1]
    src_p = jnp.concatenate(
        [edge_index[0], jnp.zeros((pad,), jnp.int32)]).reshape(NW, NB, KB)
    dst_p = jnp.concatenate(
        [edge_index[1], jnp.full((pad,), N, jnp.int32)]).reshape(NW, NB, KB)

    deg_fn = pl.kernel(
        _deg_body,
        out_type=jax.ShapeDtypeStruct((NC, NP), jnp.float32),
        mesh=_mesh,
        scratch_types=[
            pltpu.VMEM((NB, KB), jnp.int32),
            pltpu.VMEM((KB,), jnp.float32),
            pltpu.VMEM((RPT,), jnp.float32),
            pltpu.VMEM_SHARED((NP,), jnp.float32),
        ],
    )
    deg = deg_fn(dst_p)
    deg_t = deg.T

    y = pl.pallas_call(
        _mm_body,
        grid=(N // RB,),
        in_specs=[
            pl.BlockSpec((RB, D), lambda i: (i, 0)),
            pl.BlockSpec((D, D), lambda i: (0, 0)),
            pl.BlockSpec((RB, NC), lambda i: (i, 0)),
        ],
        out_specs=pl.BlockSpec((RB, D), lambda i: (i, 0)),
        out_shape=jax.ShapeDtypeStruct((N, D), jnp.float32),
    )(x, W, deg_t)

    msg_fn = pl.kernel(
        _msg_body,
        out_type=jax.ShapeDtypeStruct((NC, NP, D), jnp.float32),
        mesh=_mesh,
        scratch_types=[
            pltpu.VMEM((IR, KB), jnp.int32),
            pltpu.VMEM((IR, KB), jnp.int32),
            pltpu.VMEM((NBUF, KB, D), jnp.float32),
            pltpu.VMEM_SHARED((NP, D), jnp.float32),
            pltpu.SemaphoreType.DMA((IR,)),
            pltpu.SemaphoreType.DMA((NBUF,)),
            pltpu.SemaphoreType.DMA((NBUF,)),
        ],
    )
    acc = msg_fn(src_p, dst_p, y)

    out = pl.pallas_call(
        _fin_body,
        grid=(N // RB,),
        in_specs=[
            pl.BlockSpec((NC, RB, D), lambda i: (0, i, 0)),
            pl.BlockSpec((RB, D), lambda i: (i, 0)),
            pl.BlockSpec((RB, NC), lambda i: (i, 0)),
            pl.BlockSpec((1, D), lambda i: (0, 0)),
        ],
        out_specs=pl.BlockSpec((RB, D), lambda i: (i, 0)),
        out_shape=jax.ShapeDtypeStruct((N, D), jnp.float32),
    )(acc, y, deg_t, b.reshape(1, D))
    return out
